# baseline (device time: 99042 ns/iter reference)
import jax
import jax.numpy as jnp
from jax import lax
from jax.experimental import pallas as pl
from jax.experimental.pallas import tpu as pltpu

M = 2048
K = 8192
N = 2048
MZ = M // 2
NC = 8
CN = N // NC

BF = jnp.bfloat16
F32 = jnp.float32


def kernel(dy, W):
    my_z_outer = lax.axis_index("z")
    dy_b = lax.dynamic_slice_in_dim(dy, my_z_outer * MZ, MZ, axis=0).astype(BF)
    w_b = W.astype(BF)

    def body(dy_hbm, w_hbm, out_hbm,
             dy_bf, w_bf, p_buf, ysend, yrecv, zsend, zrecv, zf32,
             dy_sem, w_sems, ysend_sems, yrecv_sems, zsend_sems, zrecv_sems,
             oy_sems, oz_sems):
        my_x = lax.axis_index("x")
        my_y = lax.axis_index("y")
        my_z = lax.axis_index("z")
        y_nbr = (my_x, 1 - my_y, my_z)
        z_nbr = (my_x, my_y, 1 - my_z)
        row0 = my_z * MZ

        def slot2(i):
            return lax.rem(i, 2) if not isinstance(i, int) else i % 2

        def w_dma(c):
            s = slot2(c)
            return pltpu.make_async_copy(
                w_hbm.at[pl.ds(c * CN, CN), :],
                w_bf.at[pl.ds(s * CN, CN), :],
                w_sems.at[s],
            )

        def oy_dma(k):
            s = slot2(k)
            return pltpu.make_async_copy(
                p_buf.at[pl.ds(s * MZ, MZ), :],
                out_hbm.at[pl.ds(row0, MZ), pl.ds(k * CN, CN)],
                oy_sems.at[s],
            )

        def oz_dma(k):
            s = slot2(k)
            return pltpu.make_async_copy(
                zf32.at[pl.ds(s * MZ, MZ), :],
                out_hbm.at[pl.ds((1 - my_z) * MZ, MZ), pl.ds(k * CN, CN)],
                oz_sems.at[s],
            )

        def y_rdma(c):
            s = slot2(c)
            return pltpu.make_async_remote_copy(
                src_ref=ysend.at[pl.ds(s * MZ, MZ), :],
                dst_ref=yrecv.at[pl.ds(c * MZ, MZ), :],
                send_sem=ysend_sems.at[s],
                recv_sem=yrecv_sems.at[c],
                device_id=y_nbr,
                device_id_type=pl.DeviceIdType.MESH,
            )

        def z_rdma(c):
            s = slot2(c)
            return pltpu.make_async_remote_copy(
                src_ref=zsend.at[pl.ds(s * MZ, MZ), :],
                dst_ref=zrecv.at[pl.ds(c * MZ, MZ), :],
                send_sem=zsend_sems.at[s],
                recv_sem=zrecv_sems.at[c],
                device_id=z_nbr,
                device_id_type=pl.DeviceIdType.MESH,
            )

        dy_copy = pltpu.make_async_copy(dy_hbm, dy_bf, dy_sem)
        dy_copy.start()
        w_dma(0).start()
        w_dma(1).start()

        barrier = pltpu.get_barrier_semaphore()
        for nbr in (y_nbr, z_nbr):
            pl.semaphore_signal(
                barrier, inc=1, device_id=nbr,
                device_id_type=pl.DeviceIdType.MESH,
            )
        pl.semaphore_wait(barrier, 2)

        dy_copy.wait()

        def loop_body(c, carry):
            s = lax.rem(c, 2)

            @pl.when(c < NC)
            def _compute():
                w_dma(c).wait()

                @pl.when(c >= 2)
                def _():
                    oy_dma(c - 2).wait()
                    y_rdma(c - 2).wait_send()

                p = lax.dot_general(
                    dy_bf[...], w_bf[pl.ds(s * CN, CN), :],
                    dimension_numbers=(((1,), (1,)), ((), ())),
                    preferred_element_type=F32,
                )

                @pl.when(c + 2 < NC)
                def _():
                    w_dma(c + 2).start()

                p_buf[pl.ds(s * MZ, MZ), :] = p
                ysend[pl.ds(s * MZ, MZ), :] = p.astype(BF)
                y_rdma(c).start()

            @pl.when((c >= 1) & (c <= NC))
            def _reduce():
                k = c - 1
                sk = lax.rem(k, 2)
                y_rdma(k).wait_recv()
                p_buf[pl.ds(sk * MZ, MZ), :] = (
                    p_buf[pl.ds(sk * MZ, MZ), :]
                    + yrecv[pl.ds(k * MZ, MZ), :].astype(F32)
                )

                @pl.when(k >= 2)
                def _():
                    z_rdma(k - 2).wait_send()

                zsend[pl.ds(sk * MZ, MZ), :] = (
                    p_buf[pl.ds(sk * MZ, MZ), :].astype(BF)
                )
                z_rdma(k).start()
                oy_dma(k).start()

            @pl.when(c >= 2)
            def _gather():
                k = c - 2
                sk = lax.rem(k, 2)
                z_rdma(k).wait_recv()

                @pl.when(k >= 2)
                def _():
                    oz_dma(k - 2).wait()

                zf32[pl.ds(sk * MZ, MZ), :] = (
                    zrecv[pl.ds(k * MZ, MZ), :].astype(F32)
                )
                oz_dma(k).start()

            return carry

        lax.fori_loop(0, NC + 2, loop_body, 0)

        for k in (NC - 2, NC - 1):
            y_rdma(k).wait_send()
            z_rdma(k).wait_send()
            oy_dma(k).wait()
            oz_dma(k).wait()

    return pl.pallas_call(
        body,
        out_shape=jax.ShapeDtypeStruct((M, N), F32),
        in_specs=[
            pl.BlockSpec(memory_space=pl.ANY),
            pl.BlockSpec(memory_space=pl.ANY),
        ],
        out_specs=pl.BlockSpec(memory_space=pl.ANY),
        scratch_shapes=[
            pltpu.VMEM((MZ, K), BF),
            pltpu.VMEM((2 * CN, K), BF),
            pltpu.VMEM((2 * MZ, CN), F32),
            pltpu.VMEM((2 * MZ, CN), BF),
            pltpu.VMEM((NC * MZ, CN), BF),
            pltpu.VMEM((2 * MZ, CN), BF),
            pltpu.VMEM((NC * MZ, CN), BF),
            pltpu.VMEM((2 * MZ, CN), F32),
            pltpu.SemaphoreType.DMA,
            pltpu.SemaphoreType.DMA((2,)),
            pltpu.SemaphoreType.DMA((2,)),
            pltpu.SemaphoreType.DMA((NC,)),
            pltpu.SemaphoreType.DMA((2,)),
            pltpu.SemaphoreType.DMA((NC,)),
            pltpu.SemaphoreType.DMA((2,)),
            pltpu.SemaphoreType.DMA((2,)),
        ],
        compiler_params=pltpu.CompilerParams(
            collective_id=0,
            vmem_limit_bytes=63 * 1024 * 1024,
        ),
    )(dy_b, w_b)
